# Initial kernel scaffold; baseline (speedup 1.0000x reference)
import jax
import jax.numpy as jnp
from jax.experimental import pallas as pl


def kernel(channel_indices, physics_codebook, learned_embeddings):
    raise NotImplementedError("write your pallas kernel here")


# speedup vs baseline: 4.3029x; 4.3029x over previous
import functools

import jax
import jax.numpy as jnp
from jax import lax
from jax.experimental import pallas as pl
from jax.experimental.pallas import tpu as pltpu
from jax.experimental.pallas import tpu_sc as plsc

OUT_DIM = 32
NUM_CHANNELS = 512
LEARNABLE_ROWS = (0, 77, 100, 200, 300, 333, 400, 500)

LANES = 16
SUB = 8


def _merge_body(cb_ref, le_ref, out_ref):
    row = lax.broadcasted_iota(jnp.int32, (NUM_CHANNELS, OUT_DIM), 0)
    merged = cb_ref[...]
    for i, g in enumerate(LEARNABLE_ROWS):
        merged = jnp.where(row == g, le_ref[i, :][None, :], merged)
    out_ref[...] = merged


def _merge_table(codebook, learned):
    return pl.pallas_call(
        _merge_body,
        out_shape=jax.ShapeDtypeStruct((NUM_CHANNELS, OUT_DIM), jnp.float32),
    )(codebook, learned)


@functools.lru_cache(maxsize=None)
def _make_gather(n0: int, n1: int):
    info = plsc.get_sparse_core_info()
    num_workers = info.num_cores * info.num_subcores
    d_blocks = OUT_DIM // SUB
    n_tasks = n1 * d_blocks
    assert n_tasks % num_workers == 0 and n0 % 128 == 0
    tasks_per_w = n_tasks // num_workers
    tc_blocks = n0 // 128
    mesh = plsc.VectorSubcoreMesh(core_axis_name="c", subcore_axis_name="s")

    @functools.partial(
        pl.kernel,
        mesh=mesh,
        out_type=jax.ShapeDtypeStruct((n_tasks, tc_blocks, SUB, 128), jnp.float32),
        scratch_types=[
            pltpu.VMEM((NUM_CHANNELS * OUT_DIM,), jnp.float32),
            pltpu.VMEM((n0,), jnp.int32),
            pltpu.VMEM((n0,), jnp.int32),
            pltpu.VMEM((tc_blocks, SUB, 128), jnp.float32),
            pltpu.VMEM((tc_blocks, SUB, 128), jnp.float32),
            pltpu.SemaphoreType.DMA,
            pltpu.SemaphoreType.DMA,
            pltpu.SemaphoreType.DMA,
            pltpu.SemaphoreType.DMA,
        ],
        compiler_params=pltpu.CompilerParams(needs_layout_passes=False),
    )
    def gather(table_hbm, idxt_hbm, out_hbm, table_v, idx_a, idx_b,
               slab_a, slab_b, isem_a, isem_b, wsem_a, wsem_b):
        wid = lax.axis_index("s") * info.num_cores + lax.axis_index("c")
        t0 = wid * tasks_per_w

        pltpu.sync_copy(table_hbm, table_v)
        pltpu.async_copy(idxt_hbm.at[t0 // d_blocks], idx_a, isem_a)
        pltpu.async_copy(idxt_hbm.at[(t0 + 1) // d_blocks], idx_b, isem_b)

        def compute(t, idx_v, slab):
            d_base = (lax.rem(t0 + t, d_blocks)) * SUB

            @pl.loop(0, tc_blocks)
            def _(tc):
                for lb in range(128 // LANES):
                    iv = idx_v[pl.ds(tc * 128 + lb * LANES, LANES)]
                    base = iv * OUT_DIM + d_base
                    for s in range(SUB):
                        v = plsc.load_gather(table_v, [base + s])
                        slab[tc, s, pl.ds(lb * LANES, LANES)] = v

        def write(t, slab, wsem):
            pltpu.async_copy(slab, out_hbm.at[t0 + t], wsem)

        def drain_w(slab, wsem):
            pltpu.make_async_copy(slab, out_hbm.at[t0], wsem).wait()

        def drain_i(idx_v, isem):
            pltpu.make_async_copy(idxt_hbm.at[t0 // d_blocks], idx_v, isem).wait()

        def prefetch(t, idx_v, isem):
            pltpu.async_copy(idxt_hbm.at[(t0 + t) // d_blocks], idx_v, isem)

        @pl.loop(0, tasks_per_w // 2)
        def _(k):
            ta = 2 * k

            @pl.when(k > 0)
            def _():
                drain_w(slab_a, wsem_a)

            drain_i(idx_a, isem_a)
            compute(ta, idx_a, slab_a)
            write(ta, slab_a, wsem_a)
            prefetch(ta + 2, idx_a, isem_a)

            @pl.when(k > 0)
            def _():
                drain_w(slab_b, wsem_b)

            drain_i(idx_b, isem_b)
            compute(ta + 1, idx_b, slab_b)
            write(ta + 1, slab_b, wsem_b)

            @pl.when(2 * k + 3 < tasks_per_w)
            def _():
                prefetch(ta + 3, idx_b, isem_b)

        t_last = tasks_per_w - 1
        drain_w(slab_a, wsem_a)
        drain_i(idx_a, isem_a)
        compute(t_last, idx_a, slab_a)
        write(t_last, slab_a, wsem_a)
        drain_w(slab_a, wsem_a)
        drain_w(slab_b, wsem_b)

    return gather


def kernel(channel_indices, physics_codebook, learned_embeddings):
    merged = _merge_table(physics_codebook, learned_embeddings)
    n0, n1 = channel_indices.shape
    idxt = channel_indices.T
    x = _make_gather(n0, n1)(merged.reshape(-1), idxt)
    d_blocks = OUT_DIM // SUB
    return (
        x.reshape(n1, d_blocks, n0 // 128, SUB, 128)
        .transpose(2, 4, 0, 1, 3)
        .reshape(n0, n1, OUT_DIM)
    )


# speedup vs baseline: 7.2526x; 1.6855x over previous
import functools

import jax
import jax.numpy as jnp
from jax import lax
from jax.experimental import pallas as pl
from jax.experimental.pallas import tpu as pltpu
from jax.experimental.pallas import tpu_sc as plsc

OUT_DIM = 32
NUM_CHANNELS = 512
LEARNABLE_ROWS = (0, 77, 100, 200, 300, 333, 400, 500)

LANES = 16
SUB = 8


def _merge_body(cb_ref, le_ref, out_ref):
    row = lax.broadcasted_iota(jnp.int32, (NUM_CHANNELS, OUT_DIM), 0)
    merged = cb_ref[...]
    for i, g in enumerate(LEARNABLE_ROWS):
        merged = jnp.where(row == g, le_ref[i, :][None, :], merged)
    out_ref[...] = merged


def _merge_table(codebook, learned):
    return pl.pallas_call(
        _merge_body,
        out_shape=jax.ShapeDtypeStruct((NUM_CHANNELS, OUT_DIM), jnp.float32),
    )(codebook, learned)


@functools.lru_cache(maxsize=None)
def _make_gather(n0: int, n1: int):
    info = plsc.get_sparse_core_info()
    num_workers = info.num_cores * info.num_subcores
    d_blocks = OUT_DIM // SUB
    n_tasks = n1 * d_blocks
    assert n_tasks % num_workers == 0 and n0 % 128 == 0
    tasks_per_w = n_tasks // num_workers
    tc_blocks = n0 // 128
    mesh = plsc.VectorSubcoreMesh(core_axis_name="c", subcore_axis_name="s")

    @functools.partial(
        pl.kernel,
        mesh=mesh,
        out_type=jax.ShapeDtypeStruct((n_tasks, tc_blocks, SUB, 128), jnp.float32),
        scratch_types=[
            pltpu.VMEM((NUM_CHANNELS * OUT_DIM,), jnp.float32),
            pltpu.VMEM((n0,), jnp.int32),
            pltpu.VMEM((n0,), jnp.int32),
            pltpu.VMEM((tc_blocks, SUB, 128), jnp.float32),
            pltpu.VMEM((tc_blocks, SUB, 128), jnp.float32),
            pltpu.SemaphoreType.DMA,
            pltpu.SemaphoreType.DMA,
            pltpu.SemaphoreType.DMA,
            pltpu.SemaphoreType.DMA,
        ],
        compiler_params=pltpu.CompilerParams(needs_layout_passes=False),
    )
    def gather(table_hbm, idxt_hbm, out_hbm, table_v, idx_a, idx_b,
               slab_a, slab_b, isem_a, isem_b, wsem_a, wsem_b):
        wid = lax.axis_index("s") * info.num_cores + lax.axis_index("c")
        t0 = wid * tasks_per_w

        pltpu.sync_copy(table_hbm, table_v)
        pltpu.async_copy(idxt_hbm.at[t0 // d_blocks], idx_a, isem_a)
        pltpu.async_copy(idxt_hbm.at[(t0 + 1) // d_blocks], idx_b, isem_b)

        def compute(t, idx_v, slab):
            d_base = (lax.rem(t0 + t, d_blocks)) * SUB
            d_vec = jnp.full((LANES,), OUT_DIM, jnp.int32) * 0 + d_base

            @plsc.parallel_loop(0, tc_blocks, unroll=4)
            def _(tc):
                for lb in range(128 // LANES):
                    iv = idx_v[pl.ds(tc * 128 + lb * LANES, LANES)]
                    base = iv * OUT_DIM + d_vec
                    for s in range(SUB):
                        v = plsc.load_gather(table_v, [base + s])
                        slab[tc, s, pl.ds(lb * LANES, LANES)] = v

        def write(t, slab, wsem):
            pltpu.async_copy(slab, out_hbm.at[t0 + t], wsem)

        def drain_w(slab, wsem):
            pltpu.make_async_copy(slab, out_hbm.at[t0], wsem).wait()

        def drain_i(idx_v, isem):
            pltpu.make_async_copy(idxt_hbm.at[t0 // d_blocks], idx_v, isem).wait()

        def prefetch(t, idx_v, isem):
            pltpu.async_copy(idxt_hbm.at[(t0 + t) // d_blocks], idx_v, isem)

        @pl.loop(0, tasks_per_w // 2)
        def _(k):
            ta = 2 * k

            @pl.when(k > 0)
            def _():
                drain_w(slab_a, wsem_a)

            drain_i(idx_a, isem_a)
            compute(ta, idx_a, slab_a)
            write(ta, slab_a, wsem_a)
            prefetch(ta + 2, idx_a, isem_a)

            @pl.when(k > 0)
            def _():
                drain_w(slab_b, wsem_b)

            drain_i(idx_b, isem_b)
            compute(ta + 1, idx_b, slab_b)
            write(ta + 1, slab_b, wsem_b)

            @pl.when(2 * k + 3 < tasks_per_w)
            def _():
                prefetch(ta + 3, idx_b, isem_b)

        t_last = tasks_per_w - 1
        drain_w(slab_a, wsem_a)
        drain_i(idx_a, isem_a)
        compute(t_last, idx_a, slab_a)
        write(t_last, slab_a, wsem_a)
        drain_w(slab_a, wsem_a)
        drain_w(slab_b, wsem_b)

    return gather


def kernel(channel_indices, physics_codebook, learned_embeddings):
    merged = _merge_table(physics_codebook, learned_embeddings)
    n0, n1 = channel_indices.shape
    idxt = channel_indices.T
    x = _make_gather(n0, n1)(merged.reshape(-1), idxt)
    d_blocks = OUT_DIM // SUB
    return (
        x.reshape(n1, d_blocks, n0 // 128, SUB, 128)
        .transpose(2, 4, 0, 1, 3)
        .reshape(n0, n1, OUT_DIM)
    )


# speedup vs baseline: 26.6858x; 3.6795x over previous
import functools

import jax
import jax.numpy as jnp
from jax import lax
from jax.experimental import pallas as pl
from jax.experimental.pallas import tpu as pltpu
from jax.experimental.pallas import tpu_sc as plsc

OUT_DIM = 32
NUM_CHANNELS = 512
LEARNABLE_ROWS = (0, 77, 100, 200, 300, 333, 400, 500)

LANES = 16
SUB = 8


def _merge_body(cb_ref, le_ref, out_ref):
    col = lax.broadcasted_iota(jnp.int32, (OUT_DIM, NUM_CHANNELS), 1)
    merged = cb_ref[...].T
    for i, g in enumerate(LEARNABLE_ROWS):
        merged = jnp.where(col == g, le_ref[i, :][:, None], merged)
    out_ref[...] = merged


def _merge_table(codebook, learned):
    return pl.pallas_call(
        _merge_body,
        out_shape=jax.ShapeDtypeStruct((OUT_DIM, NUM_CHANNELS), jnp.float32),
    )(codebook, learned)


@functools.lru_cache(maxsize=None)
def _make_gather(n0: int, n1: int):
    info = plsc.get_sparse_core_info()
    num_workers = info.num_cores * info.num_subcores
    d_blocks = OUT_DIM // SUB
    n_tasks = n1 * d_blocks
    assert n_tasks % num_workers == 0 and n0 % 128 == 0
    tasks_per_w = n_tasks // num_workers
    tc_blocks = n0 // 128
    mesh = plsc.VectorSubcoreMesh(core_axis_name="c", subcore_axis_name="s")

    @functools.partial(
        pl.kernel,
        mesh=mesh,
        out_type=jax.ShapeDtypeStruct((n_tasks, tc_blocks, SUB, 128), jnp.float32),
        scratch_types=[
            pltpu.VMEM((NUM_CHANNELS * OUT_DIM,), jnp.float32),
            pltpu.VMEM((n0,), jnp.int32),
            pltpu.VMEM((n0,), jnp.int32),
            pltpu.VMEM((tc_blocks, SUB, 128), jnp.float32),
            pltpu.VMEM((tc_blocks, SUB, 128), jnp.float32),
            pltpu.SemaphoreType.DMA,
            pltpu.SemaphoreType.DMA,
            pltpu.SemaphoreType.DMA,
            pltpu.SemaphoreType.DMA,
        ],
        compiler_params=pltpu.CompilerParams(needs_layout_passes=False),
    )
    def gather(table_hbm, idxt_hbm, out_hbm, table_v, idx_a, idx_b,
               slab_a, slab_b, isem_a, isem_b, wsem_a, wsem_b):
        wid = lax.axis_index("s") * info.num_cores + lax.axis_index("c")
        t0 = wid * tasks_per_w

        pltpu.sync_copy(table_hbm, table_v)
        pltpu.async_copy(idxt_hbm.at[t0 // d_blocks], idx_a, isem_a)
        pltpu.async_copy(idxt_hbm.at[(t0 + 1) // d_blocks], idx_b, isem_b)

        def compute(t, idx_v, slab):
            d_base = (lax.rem(t0 + t, d_blocks)) * SUB
            base_vec = jnp.full((LANES,), 0, jnp.int32) + d_base * NUM_CHANNELS

            @plsc.parallel_loop(0, tc_blocks, unroll=4)
            def _(tc):
                for lb in range(128 // LANES):
                    iv = idx_v[pl.ds(tc * 128 + lb * LANES, LANES)]
                    base = iv + base_vec
                    for s in range(SUB):
                        v = plsc.load_gather(table_v, [base + s * NUM_CHANNELS])
                        slab[tc, s, pl.ds(lb * LANES, LANES)] = v

        def write(t, slab, wsem):
            pltpu.async_copy(slab, out_hbm.at[t0 + t], wsem)

        def drain_w(slab, wsem):
            pltpu.make_async_copy(slab, out_hbm.at[t0], wsem).wait()

        def drain_i(idx_v, isem):
            pltpu.make_async_copy(idxt_hbm.at[t0 // d_blocks], idx_v, isem).wait()

        def prefetch(t, idx_v, isem):
            pltpu.async_copy(idxt_hbm.at[(t0 + t) // d_blocks], idx_v, isem)

        @pl.loop(0, tasks_per_w // 2)
        def _(k):
            ta = 2 * k

            @pl.when(k > 0)
            def _():
                drain_w(slab_a, wsem_a)

            drain_i(idx_a, isem_a)
            compute(ta, idx_a, slab_a)
            write(ta, slab_a, wsem_a)
            prefetch(ta + 2, idx_a, isem_a)

            @pl.when(k > 0)
            def _():
                drain_w(slab_b, wsem_b)

            drain_i(idx_b, isem_b)
            compute(ta + 1, idx_b, slab_b)
            write(ta + 1, slab_b, wsem_b)

            @pl.when(2 * k + 3 < tasks_per_w)
            def _():
                prefetch(ta + 3, idx_b, isem_b)

        t_last = tasks_per_w - 1
        drain_w(slab_a, wsem_a)
        drain_i(idx_a, isem_a)
        compute(t_last, idx_a, slab_a)
        write(t_last, slab_a, wsem_a)
        drain_w(slab_a, wsem_a)
        drain_w(slab_b, wsem_b)

    return gather


def kernel(channel_indices, physics_codebook, learned_embeddings):
    merged = _merge_table(physics_codebook, learned_embeddings)
    n0, n1 = channel_indices.shape
    idxt = channel_indices.T
    x = _make_gather(n0, n1)(merged.reshape(-1), idxt)
    d_blocks = OUT_DIM // SUB
    return (
        x.reshape(n1, d_blocks, n0 // 128, SUB, 128)
        .transpose(2, 4, 0, 1, 3)
        .reshape(n0, n1, OUT_DIM)
    )
